# Initial kernel scaffold; baseline (speedup 1.0000x reference)
import jax
import jax.numpy as jnp
from jax.experimental import pallas as pl


def kernel(x, probe, Wq, bq, Wk, bk, Wv, bv, Wo, bo, ln_g, ln_b, gate_W, gate_b, fc1_W, fc1_b, fc2_W, fc2_b):
    raise NotImplementedError("write your pallas kernel here")


# speedup vs baseline: 1.3946x; 1.3946x over previous
import functools

import jax
import jax.numpy as jnp
from jax.experimental import pallas as pl
from jax.experimental.pallas import tpu as pltpu

B, S, D, H = 8, 512, 768, 12
T = 8
E, K = 16, 2
FF = 4 * D
DH = D // H
HT = H * T
N = B * T
CH = 768
NCH = FF // CH


def _attn_body(x_ref, probe_ref, wq_ref, bq_ref, wk_ref, bk_ref, wv_ref,
               bv_ref, wo_ref, bo_ref, lng_ref, lnb_ref, gw_ref, gb_ref,
               resid_ref, tok_ref, attnw_ref, comb_ref, load_ref, loss_ref,
               u_s, c_s, logit_s):
    b = pl.program_id(0)

    @pl.when(b == 0)
    def _prologue():
        p = probe_ref[0]
        q_full = jax.lax.dot_general(
            p, wq_ref[...], (((1,), (0,)), ((), ()))) + bq_ref[...]
        q_rep = jnp.broadcast_to(q_full[None], (H, T, D)).reshape(HT, D)
        row_h = jax.lax.broadcasted_iota(jnp.int32, (HT, D), 0) // T
        col_h = jax.lax.broadcasted_iota(jnp.int32, (HT, D), 1) // DH
        q_exp = jnp.where(row_h == col_h, q_rep, 0.0)
        u_s[...] = jax.lax.dot_general(
            q_exp, wk_ref[...], (((1,), (1,)), ((), ())))
        c = jnp.sum(q_exp * bk_ref[...], axis=1, keepdims=True)
        c_s[...] = jnp.broadcast_to(c, (HT, 128))

    x_b = x_ref[0]
    scale = 1.0 / jnp.sqrt(jnp.float32(DH))
    st = (jax.lax.dot_general(u_s[...], x_b, (((1,), (1,)), ((), ())))
          + c_s[:, :1]) * scale
    m = jnp.max(st, axis=1, keepdims=True)
    ex = jnp.exp(st - m)
    w = ex / jnp.sum(ex, axis=1, keepdims=True)
    attnw_ref[...] = w.reshape(1, H, T, S)

    pooled = jax.lax.dot_general(w, x_b, (((1,), (0,)), ((), ())))
    z = jax.lax.dot_general(pooled, wv_ref[...], (((1,), (0,)), ((), ())))
    z3 = z.reshape(H, T, D)
    hsel = (jax.lax.broadcasted_iota(jnp.int32, (H, T, D), 0)
            == jax.lax.broadcasted_iota(jnp.int32, (H, T, D), 2) // DH)
    ctx = jnp.sum(jnp.where(hsel, z3, 0.0), axis=0) + bv_ref[...]

    attn_out = jax.lax.dot_general(
        ctx, wo_ref[...], (((1,), (0,)), ((), ()))) + bo_ref[...]
    resid_ref[...] = attn_out[None]

    mu = jnp.mean(attn_out, axis=1, keepdims=True)
    dev = attn_out - mu
    var = jnp.mean(dev * dev, axis=1, keepdims=True)
    tok = dev * jax.lax.rsqrt(var + 1e-5) * lng_ref[...] + lnb_ref[...]
    tok_ref[...] = tok[None]

    logits = jax.lax.dot_general(
        tok, gw_ref[...], (((1,), (0,)), ((), ()))) + gb_ref[...]
    logit_s[pl.ds(b * T, T), :] = logits

    @pl.when(b == B - 1)
    def _routing():
        lg = logit_s[...]
        mm = jnp.max(lg, axis=1, keepdims=True)
        el = jnp.exp(lg - mm)
        probs = el / jnp.sum(el, axis=1, keepdims=True)
        iota = jax.lax.broadcasted_iota(jnp.int32, (N, E), 1)
        v1 = jnp.max(probs, axis=1, keepdims=True)
        i1 = jnp.min(jnp.where(probs == v1, iota, E), axis=1, keepdims=True)
        p2 = jnp.where(iota == i1, -1.0, probs)
        v2 = jnp.max(p2, axis=1, keepdims=True)
        i2 = jnp.min(jnp.where(p2 == v2, iota, E), axis=1, keepdims=True)
        denom = v1 + v2
        comb = (jnp.where(iota == i1, v1 / denom, 0.0)
                + jnp.where(iota == i2, v2 / denom, 0.0))
        comb_ref[...] = comb
        mask = (jnp.where(iota == i1, 1.0, 0.0)
                + jnp.where(iota == i2, 1.0, 0.0))
        load = jnp.sum(mask, axis=0, keepdims=True)
        load_ref[...] = load
        pbar = jnp.sum(probs, axis=0, keepdims=True) / N
        loss = E * jnp.sum((load / N) * pbar)
        loss_ref[...] = jnp.full((1, E), loss, jnp.float32)


def _ffn_body(tok_ref, comb_ref, resid_ref, w1_ref, b1_ref, w2_ref, b2_ref,
              out_ref):
    e = pl.program_id(0)
    c = pl.program_id(1)

    @pl.when((e == 0) & (c == 0))
    def _init():
        out_ref[...] = resid_ref[...]

    onehot = (jax.lax.broadcasted_iota(jnp.int32, (E, 1), 0) == e
              ).astype(jnp.float32)
    comb = jax.lax.dot_general(
        comb_ref[...], onehot, (((1,), (0,)), ((), ())))

    h = jax.lax.dot_general(
        tok_ref[...], w1_ref[0], (((1,), (0,)), ((), ()))) + b1_ref[0]
    g = jax.nn.gelu(h) * comb

    @pl.when(c == 0)
    def _bias2():
        out_ref[...] += comb * b2_ref[0]

    out_ref[...] += jax.lax.dot_general(
        g, w2_ref[0], (((1,), (0,)), ((), ())))


def kernel(x, probe, Wq, bq, Wk, bk, Wv, bv, Wo, bo, ln_g, ln_b,
           gate_W, gate_b, fc1_W, fc1_b, fc2_W, fc2_b):
    f32 = jnp.float32
    row = lambda v: v.reshape(1, -1)

    attn = pl.pallas_call(
        _attn_body,
        grid=(B,),
        in_specs=[
            pl.BlockSpec((1, S, D), lambda b: (b, 0, 0)),
            pl.BlockSpec((1, T, D), lambda b: (0, 0, 0)),
            pl.BlockSpec((D, D), lambda b: (0, 0)),
            pl.BlockSpec((1, D), lambda b: (0, 0)),
            pl.BlockSpec((D, D), lambda b: (0, 0)),
            pl.BlockSpec((1, D), lambda b: (0, 0)),
            pl.BlockSpec((D, D), lambda b: (0, 0)),
            pl.BlockSpec((1, D), lambda b: (0, 0)),
            pl.BlockSpec((D, D), lambda b: (0, 0)),
            pl.BlockSpec((1, D), lambda b: (0, 0)),
            pl.BlockSpec((1, D), lambda b: (0, 0)),
            pl.BlockSpec((1, D), lambda b: (0, 0)),
            pl.BlockSpec((D, E), lambda b: (0, 0)),
            pl.BlockSpec((1, E), lambda b: (0, 0)),
        ],
        out_specs=[
            pl.BlockSpec((1, T, D), lambda b: (b, 0, 0)),
            pl.BlockSpec((1, T, D), lambda b: (b, 0, 0)),
            pl.BlockSpec((1, H, T, S), lambda b: (b, 0, 0, 0)),
            pl.BlockSpec((N, E), lambda b: (0, 0)),
            pl.BlockSpec((1, E), lambda b: (0, 0)),
            pl.BlockSpec((1, E), lambda b: (0, 0)),
        ],
        out_shape=[
            jax.ShapeDtypeStruct((B, T, D), f32),
            jax.ShapeDtypeStruct((B, T, D), f32),
            jax.ShapeDtypeStruct((B, H, T, S), f32),
            jax.ShapeDtypeStruct((N, E), f32),
            jax.ShapeDtypeStruct((1, E), f32),
            jax.ShapeDtypeStruct((1, E), f32),
        ],
        scratch_shapes=[
            pltpu.VMEM((HT, D), f32),
            pltpu.VMEM((HT, 128), f32),
            pltpu.VMEM((N, E), f32),
        ],
    )
    residual, tokens, attn_w, combine, load2, loss2 = attn(
        x, probe, Wq, row(bq), Wk, row(bk), Wv, row(bv), Wo, row(bo),
        row(ln_g), row(ln_b), gate_W, row(gate_b))

    ffn = pl.pallas_call(
        _ffn_body,
        grid=(E, NCH),
        in_specs=[
            pl.BlockSpec((N, D), lambda e, c: (0, 0)),
            pl.BlockSpec((N, E), lambda e, c: (0, 0)),
            pl.BlockSpec((N, D), lambda e, c: (0, 0)),
            pl.BlockSpec((1, D, CH), lambda e, c: (e, 0, c)),
            pl.BlockSpec((1, 1, CH), lambda e, c: (e, 0, c)),
            pl.BlockSpec((1, CH, D), lambda e, c: (e, c, 0)),
            pl.BlockSpec((1, 1, D), lambda e, c: (e, 0, 0)),
        ],
        out_specs=pl.BlockSpec((N, D), lambda e, c: (0, 0)),
        out_shape=jax.ShapeDtypeStruct((N, D), f32),
    )
    final = ffn(tokens.reshape(N, D), combine, residual.reshape(N, D),
                fc1_W, fc1_b.reshape(E, 1, FF), fc2_W, fc2_b.reshape(E, 1, D))

    return (final.reshape(B, T, D), loss2[0, 0], load2[0], attn_w)


# speedup vs baseline: 1.5340x; 1.0999x over previous
import functools

import jax
import jax.numpy as jnp
from jax.experimental import pallas as pl
from jax.experimental.pallas import tpu as pltpu

B, S, D, H = 8, 512, 768, 12
T = 8
E, K = 16, 2
FF = 4 * D
DH = D // H
HT = H * T
N = B * T
CH = 3072
NCH = FF // CH


def _attn_body(x_ref, probe_ref, wq_ref, bq_ref, wk_ref, bk_ref, wv_ref,
               bv_ref, wo_ref, bo_ref, lng_ref, lnb_ref, gw_ref, gb_ref,
               resid_ref, tok_ref, attnw_ref, comb_ref, load_ref, loss_ref,
               u_s, c_s, logit_s):
    b = pl.program_id(0)

    @pl.when(b == 0)
    def _prologue():
        p = probe_ref[0]
        q_full = jax.lax.dot_general(
            p, wq_ref[...], (((1,), (0,)), ((), ()))) + bq_ref[...]
        q_rep = jnp.broadcast_to(q_full[None], (H, T, D)).reshape(HT, D)
        row_h = jax.lax.broadcasted_iota(jnp.int32, (HT, D), 0) // T
        col_h = jax.lax.broadcasted_iota(jnp.int32, (HT, D), 1) // DH
        q_exp = jnp.where(row_h == col_h, q_rep, 0.0)
        u_s[...] = jax.lax.dot_general(
            q_exp, wk_ref[...], (((1,), (1,)), ((), ())))
        c = jnp.sum(q_exp * bk_ref[...], axis=1, keepdims=True)
        c_s[...] = jnp.broadcast_to(c, (HT, 128))

    x_b = x_ref[0]
    scale = 1.0 / jnp.sqrt(jnp.float32(DH))
    st = (jax.lax.dot_general(u_s[...], x_b, (((1,), (1,)), ((), ())))
          + c_s[:, :1]) * scale
    m = jnp.max(st, axis=1, keepdims=True)
    ex = jnp.exp(st - m)
    w = ex / jnp.sum(ex, axis=1, keepdims=True)
    attnw_ref[...] = w.reshape(1, H, T, S)

    pooled = jax.lax.dot_general(w, x_b, (((1,), (0,)), ((), ())))
    z = jax.lax.dot_general(pooled, wv_ref[...], (((1,), (0,)), ((), ())))
    z3 = z.reshape(H, T, D)
    hsel = (jax.lax.broadcasted_iota(jnp.int32, (H, T, D), 0)
            == jax.lax.broadcasted_iota(jnp.int32, (H, T, D), 2) // DH)
    ctx = jnp.sum(jnp.where(hsel, z3, 0.0), axis=0) + bv_ref[...]

    attn_out = jax.lax.dot_general(
        ctx, wo_ref[...], (((1,), (0,)), ((), ()))) + bo_ref[...]
    resid_ref[...] = attn_out[None]

    mu = jnp.mean(attn_out, axis=1, keepdims=True)
    dev = attn_out - mu
    var = jnp.mean(dev * dev, axis=1, keepdims=True)
    tok = dev * jax.lax.rsqrt(var + 1e-5) * lng_ref[...] + lnb_ref[...]
    tok_ref[...] = tok[None]

    logits = jax.lax.dot_general(
        tok, gw_ref[...], (((1,), (0,)), ((), ()))) + gb_ref[...]
    logit_s[pl.ds(b * T, T), :] = logits

    @pl.when(b == B - 1)
    def _routing():
        lg = logit_s[...]
        mm = jnp.max(lg, axis=1, keepdims=True)
        el = jnp.exp(lg - mm)
        probs = el / jnp.sum(el, axis=1, keepdims=True)
        iota = jax.lax.broadcasted_iota(jnp.int32, (N, E), 1)
        v1 = jnp.max(probs, axis=1, keepdims=True)
        i1 = jnp.min(jnp.where(probs == v1, iota, E), axis=1, keepdims=True)
        p2 = jnp.where(iota == i1, -1.0, probs)
        v2 = jnp.max(p2, axis=1, keepdims=True)
        i2 = jnp.min(jnp.where(p2 == v2, iota, E), axis=1, keepdims=True)
        denom = v1 + v2
        comb = (jnp.where(iota == i1, v1 / denom, 0.0)
                + jnp.where(iota == i2, v2 / denom, 0.0))
        comb_ref[...] = comb
        mask = (jnp.where(iota == i1, 1.0, 0.0)
                + jnp.where(iota == i2, 1.0, 0.0))
        load = jnp.sum(mask, axis=0, keepdims=True)
        load_ref[...] = load
        pbar = jnp.sum(probs, axis=0, keepdims=True) / N
        loss = E * jnp.sum((load / N) * pbar)
        loss_ref[...] = jnp.full((1, E), loss, jnp.float32)


def _ffn_body(tok_ref, comb_ref, resid_ref, w1_ref, b1_ref, w2_ref, b2_ref,
              out_ref):
    e = pl.program_id(0)
    c = pl.program_id(1)

    @pl.when((e == 0) & (c == 0))
    def _init():
        out_ref[...] = resid_ref[...]

    onehot = (jax.lax.broadcasted_iota(jnp.int32, (E, 1), 0) == e
              ).astype(jnp.float32)
    comb = jax.lax.dot_general(
        comb_ref[...], onehot, (((1,), (0,)), ((), ())))

    h = jax.lax.dot_general(
        tok_ref[...], w1_ref[0], (((1,), (0,)), ((), ()))) + b1_ref[0]
    g = jax.nn.gelu(h) * comb

    @pl.when(c == 0)
    def _bias2():
        out_ref[...] += comb * b2_ref[0]

    out_ref[...] += jax.lax.dot_general(
        g, w2_ref[0], (((1,), (0,)), ((), ())))


def kernel(x, probe, Wq, bq, Wk, bk, Wv, bv, Wo, bo, ln_g, ln_b,
           gate_W, gate_b, fc1_W, fc1_b, fc2_W, fc2_b):
    f32 = jnp.float32
    row = lambda v: v.reshape(1, -1)

    attn = pl.pallas_call(
        _attn_body,
        grid=(B,),
        in_specs=[
            pl.BlockSpec((1, S, D), lambda b: (b, 0, 0)),
            pl.BlockSpec((1, T, D), lambda b: (0, 0, 0)),
            pl.BlockSpec((D, D), lambda b: (0, 0)),
            pl.BlockSpec((1, D), lambda b: (0, 0)),
            pl.BlockSpec((D, D), lambda b: (0, 0)),
            pl.BlockSpec((1, D), lambda b: (0, 0)),
            pl.BlockSpec((D, D), lambda b: (0, 0)),
            pl.BlockSpec((1, D), lambda b: (0, 0)),
            pl.BlockSpec((D, D), lambda b: (0, 0)),
            pl.BlockSpec((1, D), lambda b: (0, 0)),
            pl.BlockSpec((1, D), lambda b: (0, 0)),
            pl.BlockSpec((1, D), lambda b: (0, 0)),
            pl.BlockSpec((D, E), lambda b: (0, 0)),
            pl.BlockSpec((1, E), lambda b: (0, 0)),
        ],
        out_specs=[
            pl.BlockSpec((1, T, D), lambda b: (b, 0, 0)),
            pl.BlockSpec((1, T, D), lambda b: (b, 0, 0)),
            pl.BlockSpec((1, H, T, S), lambda b: (b, 0, 0, 0)),
            pl.BlockSpec((N, E), lambda b: (0, 0)),
            pl.BlockSpec((1, E), lambda b: (0, 0)),
            pl.BlockSpec((1, E), lambda b: (0, 0)),
        ],
        out_shape=[
            jax.ShapeDtypeStruct((B, T, D), f32),
            jax.ShapeDtypeStruct((B, T, D), f32),
            jax.ShapeDtypeStruct((B, H, T, S), f32),
            jax.ShapeDtypeStruct((N, E), f32),
            jax.ShapeDtypeStruct((1, E), f32),
            jax.ShapeDtypeStruct((1, E), f32),
        ],
        scratch_shapes=[
            pltpu.VMEM((HT, D), f32),
            pltpu.VMEM((HT, 128), f32),
            pltpu.VMEM((N, E), f32),
        ],
    )
    residual, tokens, attn_w, combine, load2, loss2 = attn(
        x, probe, Wq, row(bq), Wk, row(bk), Wv, row(bv), Wo, row(bo),
        row(ln_g), row(ln_b), gate_W, row(gate_b))

    ffn = pl.pallas_call(
        _ffn_body,
        grid=(E, NCH),
        in_specs=[
            pl.BlockSpec((N, D), lambda e, c: (0, 0)),
            pl.BlockSpec((N, E), lambda e, c: (0, 0)),
            pl.BlockSpec((N, D), lambda e, c: (0, 0)),
            pl.BlockSpec((1, D, CH), lambda e, c: (e, 0, c)),
            pl.BlockSpec((1, 1, CH), lambda e, c: (e, 0, c)),
            pl.BlockSpec((1, CH, D), lambda e, c: (e, c, 0)),
            pl.BlockSpec((1, 1, D), lambda e, c: (e, 0, 0)),
        ],
        out_specs=pl.BlockSpec((N, D), lambda e, c: (0, 0)),
        out_shape=jax.ShapeDtypeStruct((N, D), f32),
    )
    final = ffn(tokens.reshape(N, D), combine, residual.reshape(N, D),
                fc1_W, fc1_b.reshape(E, 1, FF), fc2_W, fc2_b.reshape(E, 1, D))

    return (final.reshape(B, T, D), loss2[0, 0], load2[0], attn_w)
